# baseline (speedup 1.0000x reference)
import functools

import jax
import jax.numpy as jnp
from jax import lax
from jax.experimental import pallas as pl
from jax.experimental.pallas import tpu as pltpu
from jax.experimental.pallas import tpu_sc as plsc

_B, _N, _D, _K = 16, 576, 64, 1024
_T = _B * _N
_BLK = 512
_G = _T // _BLK

_NC, _NS = 2, 16
_NW = _NC * _NS
_BPW = _T // _NW
_CHUNK = 96
_NCHUNK = _BPW // _CHUNK


def _argmin_body(x_ref, w_ref, idx_ref):
    x = x_ref[...]
    w = w_ref[...]
    x2 = jnp.sum(x * x, axis=1, keepdims=True)
    w2 = jnp.sum(w * w, axis=1)
    dot = lax.dot_general(x, w, (((1,), (1,)), ((), ())),
                          preferred_element_type=jnp.float32)
    d2 = x2 + w2[None, :] - 2.0 * dot
    dist = jnp.sqrt(jnp.maximum(d2, 0.0))
    idx_ref[0, 0, :] = jnp.argmin(dist, axis=1).astype(jnp.int32)


_argmin_call = pl.pallas_call(
    _argmin_body,
    grid=(_G,),
    in_specs=[
        pl.BlockSpec((_BLK, _D), lambda i: (i, 0)),
        pl.BlockSpec((_K, _D), lambda i: (0, 0)),
    ],
    out_specs=pl.BlockSpec((1, 1, _BLK), lambda i: (i, 0, 0)),
    out_shape=jax.ShapeDtypeStruct((_G, 1, _BLK), jnp.int32),
)


def _gather_body(w_hbm, idx_hbm, out_hbm, idx_v, rows_v, sem):
    wid = lax.axis_index("s") * _NC + lax.axis_index("c")
    base = wid * _BPW
    pltpu.sync_copy(idx_hbm.at[pl.ds(base, _BPW)], idx_v)
    copies = [
        pltpu.async_copy(
            w_hbm.at[idx_v.at[pl.ds(j * _CHUNK, _CHUNK)]],
            rows_v.at[pl.ds(j * _CHUNK, _CHUNK)],
            sem,
        )
        for j in range(_NCHUNK)
    ]
    for c in copies:
        c.wait()
    pltpu.sync_copy(rows_v, out_hbm.at[pl.ds(base, _BPW)])


_gather_call = pl.kernel(
    _gather_body,
    out_type=jax.ShapeDtypeStruct((_T, _D), jnp.float32),
    mesh=plsc.VectorSubcoreMesh(core_axis_name="c", subcore_axis_name="s"),
    scratch_types=[
        pltpu.VMEM((_BPW,), jnp.int32),
        pltpu.VMEM((_BPW, _D), jnp.float32),
        pltpu.SemaphoreType.DMA,
    ],
    compiler_params=pltpu.CompilerParams(use_tc_tiling_on_sc=False),
)


def kernel(x, W):
    xf = x.reshape(_T, _D)
    idx = _argmin_call(xf, W).reshape(_T)
    quantized = _gather_call(W, idx).reshape(_B, _N, _D)
    return quantized, idx.reshape(_B, _N)


# speedup vs baseline: 1.0006x; 1.0006x over previous
import functools

import jax
import jax.numpy as jnp
from jax import lax
from jax.experimental import pallas as pl
from jax.experimental.pallas import tpu as pltpu
from jax.experimental.pallas import tpu_sc as plsc

_B, _N, _D, _K = 16, 576, 64, 1024
_T = _B * _N
_BLK = 1024
_G = _T // _BLK

_NC, _NS = 2, 16
_NW = _NC * _NS
_BPW = _T // _NW
_CHUNK = 96
_NCHUNK = _BPW // _CHUNK


def _next_f32(t):
    return lax.bitcast_convert_type(
        lax.bitcast_convert_type(t, jnp.uint32) + jnp.uint32(1), jnp.float32)


def _argmin_body(x_ref, w_ref, idx_ref):
    x = x_ref[...]
    w = w_ref[...]
    x2 = jnp.sum(x * x, axis=1, keepdims=True)
    w2 = jnp.sum(w * w, axis=1)
    ndot = lax.dot_general(-2.0 * x, w, (((1,), (1,)), ((), ())),
                           preferred_element_type=jnp.float32)
    c = jnp.maximum((x2 + w2[None, :]) + ndot, 0.0)
    m = jnp.min(c, axis=1, keepdims=True)
    mr = m.reshape(_BLK)
    u = jnp.sqrt(mr)
    t = mr
    for _ in range(5):
        t1 = _next_f32(t)
        t = jnp.where(jnp.sqrt(t1) <= u, t1, t)
    tau = t.reshape(_BLK, 1)
    ii = lax.broadcasted_iota(jnp.int32, c.shape, 1).astype(jnp.float32)
    cand = jnp.where(c <= tau, ii, float(_K))
    idx_ref[0, 0, :] = jnp.min(cand, axis=1).astype(jnp.int32)


_argmin_call = pl.pallas_call(
    _argmin_body,
    grid=(_G,),
    in_specs=[
        pl.BlockSpec((_BLK, _D), lambda i: (i, 0)),
        pl.BlockSpec((_K, _D), lambda i: (0, 0)),
    ],
    out_specs=pl.BlockSpec((1, 1, _BLK), lambda i: (i, 0, 0)),
    out_shape=jax.ShapeDtypeStruct((_G, 1, _BLK), jnp.int32),
)


def _gather_body(w_hbm, idx_hbm, out_hbm, idx_v, rows_v, sem):
    wid = lax.axis_index("s") * _NC + lax.axis_index("c")
    base = wid * _BPW
    pltpu.sync_copy(idx_hbm.at[pl.ds(base, _BPW)], idx_v)
    copies = [
        pltpu.async_copy(
            w_hbm.at[idx_v.at[pl.ds(j * _CHUNK, _CHUNK)]],
            rows_v.at[pl.ds(j * _CHUNK, _CHUNK)],
            sem,
        )
        for j in range(_NCHUNK)
    ]
    for c in copies:
        c.wait()
    pltpu.sync_copy(rows_v, out_hbm.at[pl.ds(base, _BPW)])


@functools.lru_cache(maxsize=None)
def _make_gather_call():
    return pl.kernel(
        _gather_body,
        out_type=jax.ShapeDtypeStruct((_T, _D), jnp.float32),
        mesh=plsc.VectorSubcoreMesh(core_axis_name="c", subcore_axis_name="s"),
        scratch_types=[
            pltpu.VMEM((_BPW,), jnp.int32),
            pltpu.VMEM((_BPW, _D), jnp.float32),
            pltpu.SemaphoreType.DMA,
        ],
        compiler_params=pltpu.CompilerParams(use_tc_tiling_on_sc=False),
    )


def kernel(x, W):
    xf = x.reshape(_T, _D)
    idx = _argmin_call(xf, W).reshape(_T)
    quantized = _make_gather_call()(W, idx).reshape(_B, _N, _D)
    return quantized, idx.reshape(_B, _N)


# speedup vs baseline: 1.0489x; 1.0484x over previous
import functools

import jax
import jax.numpy as jnp
from jax import lax
from jax.experimental import pallas as pl
from jax.experimental.pallas import tpu as pltpu
from jax.experimental.pallas import tpu_sc as plsc

_B, _N, _D, _K = 16, 576, 64, 1024
_T = _B * _N
_BROWS = 2
_BLK = _BROWS * _N
_G = _B // _BROWS

_NC, _NS = 2, 16
_NW = _NC * _NS
_BPW = _T // _NW
_CHUNK = 96
_NCHUNK = _BPW // _CHUNK


def _next_f32(t):
    return lax.bitcast_convert_type(
        lax.bitcast_convert_type(t, jnp.uint32) + jnp.uint32(1), jnp.float32)


def _argmin_body(x_ref, w_ref, idx_ref):
    x = x_ref[...].reshape(_BLK, _D)
    w = w_ref[...]
    x2 = jnp.sum(x * x, axis=1, keepdims=True)
    w2 = jnp.sum(w * w, axis=1)
    ndot = lax.dot_general(-2.0 * x, w, (((1,), (1,)), ((), ())),
                           preferred_element_type=jnp.float32)
    d2 = (x2 + w2[None, :]) + ndot
    m = jnp.maximum(jnp.min(d2, axis=1, keepdims=True), 0.0)
    u = jnp.sqrt(m)
    t = m
    for _ in range(5):
        t1 = _next_f32(t)
        t = jnp.where(jnp.sqrt(t1) <= u, t1, t)
    ii = lax.broadcasted_iota(jnp.int32, d2.shape, 1).astype(jnp.float32)
    cand = jnp.where(d2 <= t, ii, float(_K))
    idx_ref[0, 0, :] = jnp.min(cand, axis=1).astype(jnp.int32)


_argmin_call = pl.pallas_call(
    _argmin_body,
    grid=(_G,),
    in_specs=[
        pl.BlockSpec((_BROWS, _N, _D), lambda i: (i, 0, 0)),
        pl.BlockSpec((_K, _D), lambda i: (0, 0)),
    ],
    out_specs=pl.BlockSpec((1, 1, _BLK), lambda i: (i, 0, 0)),
    out_shape=jax.ShapeDtypeStruct((_G, 1, _BLK), jnp.int32),
)


def _gather_body(w_hbm, idx_hbm, out_hbm, idx_v, rows_v, sem):
    wid = lax.axis_index("s") * _NC + lax.axis_index("c")
    base = wid * _BPW
    pltpu.sync_copy(idx_hbm.at[pl.ds(base, _BPW)], idx_v)
    copies = [
        pltpu.async_copy(
            w_hbm.at[idx_v.at[pl.ds(j * _CHUNK, _CHUNK)]],
            rows_v.at[pl.ds(j * _CHUNK, _CHUNK)],
            sem,
        )
        for j in range(_NCHUNK)
    ]
    for c in copies:
        c.wait()
    pltpu.sync_copy(rows_v, out_hbm.at[pl.ds(base, _BPW)])


@functools.lru_cache(maxsize=None)
def _make_gather_call():
    return pl.kernel(
        _gather_body,
        out_type=jax.ShapeDtypeStruct((_T, _D), jnp.float32),
        mesh=plsc.VectorSubcoreMesh(core_axis_name="c", subcore_axis_name="s"),
        scratch_types=[
            pltpu.VMEM((_BPW,), jnp.int32),
            pltpu.VMEM((_BPW, _D), jnp.float32),
            pltpu.SemaphoreType.DMA,
        ],
        compiler_params=pltpu.CompilerParams(use_tc_tiling_on_sc=False),
    )


def kernel(x, W):
    idx = _argmin_call(x, W).reshape(_T)
    quantized = _make_gather_call()(W, idx).reshape(_B, _N, _D)
    return quantized, idx.reshape(_B, _N)


# speedup vs baseline: 1.3183x; 1.2568x over previous
import functools

import jax
import jax.numpy as jnp
from jax import lax
from jax.experimental import pallas as pl
from jax.experimental.pallas import tpu as pltpu
from jax.experimental.pallas import tpu_sc as plsc

_B, _N, _D, _K = 16, 576, 64, 1024
_T = _B * _N
_BROWS = 2
_BLK = _BROWS * _N
_G = _B // _BROWS

_NC, _NS = 2, 16
_NW = _NC * _NS
_BPW = _T // _NW
_CHUNK = 96
_NCHUNK = _BPW // _CHUNK


def _next_f32(t):
    return lax.bitcast_convert_type(
        lax.bitcast_convert_type(t, jnp.uint32) + jnp.uint32(1), jnp.float32)


def _argmin_body(x_ref, w_ref, idx_ref, q_ref):
    x = x_ref[...].reshape(_BLK, _D)
    w = w_ref[...]
    x2 = jnp.sum(x * x, axis=1, keepdims=True)
    w2 = jnp.sum(w * w, axis=1)
    ndot = lax.dot_general(-2.0 * x, w, (((1,), (1,)), ((), ())),
                           preferred_element_type=jnp.float32)
    d2 = (x2 + w2[None, :]) + ndot
    m = jnp.maximum(jnp.min(d2, axis=1, keepdims=True), 0.0)
    u = jnp.sqrt(m)
    t = m
    for _ in range(5):
        t1 = _next_f32(t)
        t = jnp.where(jnp.sqrt(t1) <= u, t1, t)
    ii = lax.broadcasted_iota(jnp.int32, d2.shape, 1).astype(jnp.float32)
    cand = jnp.where(d2 <= t, ii, float(_K))
    idxf = jnp.min(cand, axis=1, keepdims=True)
    idx_ref[0, 0, :] = idxf.reshape(_BLK).astype(jnp.int32)
    onehot = jnp.where(ii == idxf, 1.0, 0.0)
    q = lax.dot_general(onehot, w, (((1,), (0,)), ((), ())),
                        preferred_element_type=jnp.float32)
    q_ref[...] = q.reshape(_BROWS, _N, _D)


_argmin_call = pl.pallas_call(
    _argmin_body,
    grid=(_G,),
    in_specs=[
        pl.BlockSpec((_BROWS, _N, _D), lambda i: (i, 0, 0)),
        pl.BlockSpec((_K, _D), lambda i: (0, 0)),
    ],
    out_specs=[
        pl.BlockSpec((1, 1, _BLK), lambda i: (i, 0, 0)),
        pl.BlockSpec((_BROWS, _N, _D), lambda i: (i, 0, 0)),
    ],
    out_shape=[
        jax.ShapeDtypeStruct((_G, 1, _BLK), jnp.int32),
        jax.ShapeDtypeStruct((_B, _N, _D), jnp.float32),
    ],
)


def _gather_body(w_hbm, idx_hbm, out_hbm, idx_v, rows_v, sem):
    wid = lax.axis_index("s") * _NC + lax.axis_index("c")
    base = wid * _BPW
    pltpu.sync_copy(idx_hbm.at[pl.ds(base, _BPW)], idx_v)
    copies = [
        pltpu.async_copy(
            w_hbm.at[idx_v.at[pl.ds(j * _CHUNK, _CHUNK)]],
            rows_v.at[pl.ds(j * _CHUNK, _CHUNK)],
            sem,
        )
        for j in range(_NCHUNK)
    ]
    for c in copies:
        c.wait()
    pltpu.sync_copy(rows_v, out_hbm.at[pl.ds(base, _BPW)])


@functools.lru_cache(maxsize=None)
def _make_gather_call():
    return pl.kernel(
        _gather_body,
        out_type=jax.ShapeDtypeStruct((_T, _D), jnp.float32),
        mesh=plsc.VectorSubcoreMesh(core_axis_name="c", subcore_axis_name="s"),
        scratch_types=[
            pltpu.VMEM((_BPW,), jnp.int32),
            pltpu.VMEM((_BPW, _D), jnp.float32),
            pltpu.SemaphoreType.DMA,
        ],
        compiler_params=pltpu.CompilerParams(use_tc_tiling_on_sc=False),
    )


def kernel(x, W):
    idx, quantized = _argmin_call(x, W)
    return quantized, idx.reshape(_B, _N)
